# baseline (speedup 1.0000x reference)
import functools

import jax
import jax.numpy as jnp
from jax import lax
from jax.experimental import pallas as pl
from jax.experimental.pallas import tpu as pltpu
from jax.experimental.pallas import tpu_sc as plsc


def _dyn_gather(vec, idx):
    dn = lax.GatherDimensionNumbers(
        offset_dims=(), collapsed_slice_dims=(0,), start_index_map=(0,))
    return lax.gather(vec, idx[:, None], dn, slice_sizes=(1,),
                      mode=lax.GatherScatterMode.PROMISE_IN_BOUNDS)


@functools.lru_cache(maxsize=None)
def _build_sc_call(B, D4, n_t, n_c, n_e, n_d):
    info = plsc.get_sparse_core_info()
    NC, NS = info.num_cores, info.num_subcores
    NW = NC * NS
    rows_w = B // NW
    CHUNK = 64
    n_chunks = rows_w // CHUNK
    CROWS = 4 * CHUNK
    G = CROWS // 128

    off_c = n_t
    off_e = n_t + n_c
    off_d = n_t + n_c + n_e

    mesh = plsc.VectorSubcoreMesh(core_axis_name="c", subcore_axis_name="s")

    @functools.partial(
        pl.kernel,
        out_type=jax.ShapeDtypeStruct((B, 4 * D4), jnp.float32),
        mesh=mesh,
        scratch_types=[
            pltpu.VMEM((rows_w,), jnp.int32),
            pltpu.VMEM((rows_w,), jnp.int32),
            pltpu.VMEM((rows_w,), jnp.int32),
            pltpu.VMEM((2, G, 128), jnp.int32),
            pltpu.VMEM((2, CROWS, D4), jnp.float32),
            pltpu.VMEM_SHARED((n_t + n_c + n_e + n_d, D4), jnp.float32),
            pltpu.SemaphoreType.DMA,
            pltpu.SemaphoreType.DMA,
            pltpu.SemaphoreType.DMA,
            pltpu.SemaphoreType.DMA,
        ],
    )
    def body(tbl_t, tbl_c, tbl_e, tbl_d, cdep, elen, dlen, out,
             cbuf, ebuf, dbuf, ixb, dest,
             shtbl, gsem0, gsem1, wsem0, wsem1):
        gsems = (gsem0, gsem1)
        wsems = (wsem0, wsem1)

        sid = lax.axis_index("s")
        wid = sid * NC + lax.axis_index("c")
        obase = wid * rows_w
        @pl.when(sid == 0)
        def _():
            pltpu.sync_copy(tbl_t, shtbl.at[pl.ds(0, n_t)])
            pltpu.sync_copy(tbl_c, shtbl.at[pl.ds(off_c, n_c)])
            pltpu.sync_copy(tbl_e, shtbl.at[pl.ds(off_e, n_e)])
            pltpu.sync_copy(tbl_d, shtbl.at[pl.ds(off_d, n_d)])

        pltpu.sync_copy(cdep.at[pl.ds(obase, rows_w)], cbuf)
        pltpu.sync_copy(elen.at[pl.ds(obase, rows_w)], ebuf)
        pltpu.sync_copy(dlen.at[pl.ds(obase, rows_w)], dbuf)
        plsc.subcore_barrier()

        lane = lax.iota(jnp.int32, 16)
        perms = tuple(lax.shift_right_logical(lane, 2) + 4 * q
                      for q in range(4))
        seg = lax.bitwise_and(lane, 3)
        masks = tuple(seg == s for s in range(4))

        def build_idx(ci):
            p = ci % 2
            g0 = obase + ci * CHUNK
            for j in range(CHUNK // 16):
                o = ci * CHUNK + j * 16
                r = g0 + (j * 16) + lane
                t = lax.rem(r, n_t)
                cv = jnp.minimum(cbuf[pl.ds(o, 16)], n_c - 1) + off_c
                ev = jnp.minimum(ebuf[pl.ds(o, 16)], n_e - 1) + off_e
                dv = jnp.minimum(dbuf[pl.ds(o, 16)], n_d - 1) + off_d
                for q in range(4):
                    pm = perms[q]
                    iv = jnp.where(
                        masks[0], _dyn_gather(t, pm),
                        jnp.where(
                            masks[1], _dyn_gather(cv, pm),
                            jnp.where(
                                masks[2], _dyn_gather(ev, pm),
                                _dyn_gather(dv, pm))))
                    pos = 64 * j + 16 * q
                    ixb[p, pos // 128, pl.ds(pos % 128, 16)] = iv

        def fire_gathers(ci):
            p = ci % 2
            return [
                pltpu.async_copy(shtbl.at[ixb.at[p, g]],
                                 dest.at[p, pl.ds(g * 128, 128)], gsems[p])
                for g in range(G)
            ]

        def fire_write(ci):
            p = ci % 2
            g0 = obase + ci * CHUNK
            return pltpu.async_copy(
                dest.at[p].reshape(CHUNK, 4 * D4),
                out.at[pl.ds(g0, CHUNK)], wsems[p])

        build_idx(0)
        ghs = [None] * n_chunks
        whs = [None] * n_chunks
        ghs[0] = fire_gathers(0)
        for ci in range(n_chunks):
            if ci + 1 < n_chunks:
                if ci >= 1:
                    whs[ci - 1].wait()
                build_idx(ci + 1)
                ghs[ci + 1] = fire_gathers(ci + 1)
            for h in ghs[ci]:
                h.wait()
            whs[ci] = fire_write(ci)
        whs[n_chunks - 2].wait()
        whs[n_chunks - 1].wait()

    return body


def kernel(pe_temporal, pe_causal, pe_epistemic, pe_deontic,
           causal_depth, epistemic_len, deontic_len):
    n_t, d4 = pe_temporal.shape
    n_c = pe_causal.shape[0]
    n_e = pe_epistemic.shape[0]
    n_d = pe_deontic.shape[0]
    B = causal_depth.shape[0]
    call = _build_sc_call(B, d4, n_t, n_c, n_e, n_d)
    return call(pe_temporal, pe_causal, pe_epistemic, pe_deontic,
                causal_depth.astype(jnp.int32),
                epistemic_len.astype(jnp.int32),
                deontic_len.astype(jnp.int32))


# speedup vs baseline: 1.0060x; 1.0060x over previous
import functools

import jax
import jax.numpy as jnp
from jax import lax
from jax.experimental import pallas as pl
from jax.experimental.pallas import tpu as pltpu
from jax.experimental.pallas import tpu_sc as plsc


def _dyn_gather(vec, idx):
    dn = lax.GatherDimensionNumbers(
        offset_dims=(), collapsed_slice_dims=(0,), start_index_map=(0,))
    return lax.gather(vec, idx[:, None], dn, slice_sizes=(1,),
                      mode=lax.GatherScatterMode.PROMISE_IN_BOUNDS)


@functools.lru_cache(maxsize=None)
def _build_sc_call(B, D4, n_t, n_c, n_e, n_d):
    info = plsc.get_sparse_core_info()
    NC, NS = info.num_cores, info.num_subcores
    NW = NC * NS
    rows_w = B // NW
    CHUNK = 64
    n_chunks = rows_w // CHUNK
    CROWS = 4 * CHUNK
    G = CROWS // 128

    off_c = n_t
    off_e = n_t + n_c
    off_d = n_t + n_c + n_e

    mesh = plsc.VectorSubcoreMesh(core_axis_name="c", subcore_axis_name="s")

    @functools.partial(
        pl.kernel,
        out_type=jax.ShapeDtypeStruct((B, 4 * D4), jnp.float32),
        mesh=mesh,
        scratch_types=[
            pltpu.VMEM((rows_w,), jnp.int32),
            pltpu.VMEM((rows_w,), jnp.int32),
            pltpu.VMEM((rows_w,), jnp.int32),
            pltpu.VMEM((2, G, 128), jnp.int32),
            pltpu.VMEM((2, CROWS, D4), jnp.float32),
            pltpu.VMEM_SHARED((n_t + n_c + n_e + n_d, D4), jnp.float32),
            pltpu.SemaphoreType.DMA,
            pltpu.SemaphoreType.DMA,
            pltpu.SemaphoreType.DMA,
            pltpu.SemaphoreType.DMA,
        ],
    )
    def body(tbl_t, tbl_c, tbl_e, tbl_d, cdep, elen, dlen, out,
             cbuf, ebuf, dbuf, ixb, dest,
             shtbl, gsem0, gsem1, wsem0, wsem1):
        gsems = (gsem0, gsem1)
        wsems = (wsem0, wsem1)

        sid = lax.axis_index("s")
        wid = lax.axis_index("c") * NS + sid
        obase = wid * rows_w
        @pl.when(sid == 0)
        def _():
            pltpu.sync_copy(tbl_t, shtbl.at[pl.ds(0, n_t)])
            pltpu.sync_copy(tbl_c, shtbl.at[pl.ds(off_c, n_c)])
            pltpu.sync_copy(tbl_e, shtbl.at[pl.ds(off_e, n_e)])
            pltpu.sync_copy(tbl_d, shtbl.at[pl.ds(off_d, n_d)])

        pltpu.sync_copy(cdep.at[pl.ds(obase, rows_w)], cbuf)
        pltpu.sync_copy(elen.at[pl.ds(obase, rows_w)], ebuf)
        pltpu.sync_copy(dlen.at[pl.ds(obase, rows_w)], dbuf)
        plsc.subcore_barrier()

        lane = lax.iota(jnp.int32, 16)
        perms = tuple(lax.shift_right_logical(lane, 2) + 4 * q
                      for q in range(4))
        seg = lax.bitwise_and(lane, 3)
        masks = tuple(seg == s for s in range(4))

        def build_idx(ci):
            p = ci % 2
            g0 = obase + ci * CHUNK
            for j in range(CHUNK // 16):
                o = ci * CHUNK + j * 16
                r = g0 + (j * 16) + lane
                t = lax.rem(r, n_t)
                cv = jnp.minimum(cbuf[pl.ds(o, 16)], n_c - 1) + off_c
                ev = jnp.minimum(ebuf[pl.ds(o, 16)], n_e - 1) + off_e
                dv = jnp.minimum(dbuf[pl.ds(o, 16)], n_d - 1) + off_d
                for q in range(4):
                    pm = perms[q]
                    iv = jnp.where(
                        masks[0], _dyn_gather(t, pm),
                        jnp.where(
                            masks[1], _dyn_gather(cv, pm),
                            jnp.where(
                                masks[2], _dyn_gather(ev, pm),
                                _dyn_gather(dv, pm))))
                    pos = 64 * j + 16 * q
                    ixb[p, pos // 128, pl.ds(pos % 128, 16)] = iv

        def fire_gathers(ci):
            p = ci % 2
            return [
                pltpu.async_copy(shtbl.at[ixb.at[p, g]],
                                 dest.at[p, pl.ds(g * 128, 128)], gsems[p])
                for g in range(G)
            ]

        def fire_write(ci):
            p = ci % 2
            g0 = obase + ci * CHUNK
            return pltpu.async_copy(
                dest.at[p].reshape(CHUNK, 4 * D4),
                out.at[pl.ds(g0, CHUNK)], wsems[p])

        build_idx(0)
        ghs = [None] * n_chunks
        whs = [None] * n_chunks
        ghs[0] = fire_gathers(0)
        for ci in range(n_chunks):
            if ci + 1 < n_chunks:
                if ci >= 1:
                    whs[ci - 1].wait()
                build_idx(ci + 1)
                ghs[ci + 1] = fire_gathers(ci + 1)
            for h in ghs[ci]:
                h.wait()
            whs[ci] = fire_write(ci)
        whs[n_chunks - 2].wait()
        whs[n_chunks - 1].wait()

    return body


def kernel(pe_temporal, pe_causal, pe_epistemic, pe_deontic,
           causal_depth, epistemic_len, deontic_len):
    n_t, d4 = pe_temporal.shape
    n_c = pe_causal.shape[0]
    n_e = pe_epistemic.shape[0]
    n_d = pe_deontic.shape[0]
    B = causal_depth.shape[0]
    call = _build_sc_call(B, d4, n_t, n_c, n_e, n_d)
    return call(pe_temporal, pe_causal, pe_epistemic, pe_deontic,
                causal_depth.astype(jnp.int32),
                epistemic_len.astype(jnp.int32),
                deontic_len.astype(jnp.int32))
